# baseline (speedup 1.0000x reference)
import functools

import jax
import jax.numpy as jnp
from jax import lax
from jax.experimental import pallas as pl
from jax.experimental.pallas import tpu as pltpu
from jax.experimental.pallas import tpu_sc as plsc

M = 1000
A = 7
D = 256
K = 3
L = 16
ROW = A * D
MP = 1024
NW = 32
RPW = MP // NW
LAST_ROWS = M - (NW - 1) * RPW
NCHUNK = D // L

_GATHER_DN = lax.GatherDimensionNumbers(
    offset_dims=(), collapsed_slice_dims=(0,), start_index_map=(0,))


def _lane_gather(x, idx):
    return lax.gather(x, idx[:, None], _GATHER_DN, slice_sizes=(1,),
                      mode=lax.GatherScatterMode.PROMISE_IN_BOUNDS)


def _splat_sum(x):
    lanes = lax.iota(jnp.int32, L)
    for step in (1, 2, 4, 8):
        x = x + _lane_gather(x, lanes ^ step)
    return x


_MESH = plsc.VectorSubcoreMesh(core_axis_name="c", subcore_axis_name="s")


@functools.partial(
    pl.kernel,
    out_type=(
        jax.ShapeDtypeStruct((MP,), jnp.float32),
        jax.ShapeDtypeStruct((MP,), jnp.float32),
    ),
    mesh=_MESH,
    scratch_types=[
        pltpu.VMEM((RPW, ROW), jnp.float32),
        pltpu.VMEM((D,), jnp.float32),
        pltpu.VMEM((RPW,), jnp.float32),
        pltpu.VMEM((RPW,), jnp.float32),
    ],
)
def _dot_kernel(q_hbm, bank_hbm, dot_hbm, sq_hbm, slab_v, q_v, dot_v, sq_v):
    wid = lax.axis_index("s") * 2 + lax.axis_index("c")
    r0 = wid * RPW
    pltpu.sync_copy(q_hbm, q_v)

    @pl.when(wid < NW - 1)
    def _():
        pltpu.sync_copy(bank_hbm.at[pl.ds(r0, RPW)], slab_v)

    @pl.when(wid == NW - 1)
    def _():
        pltpu.sync_copy(bank_hbm.at[pl.ds(r0, LAST_ROWS)],
                        slab_v.at[pl.ds(0, LAST_ROWS)])

    lanes = lax.iota(jnp.int32, L)
    for g in range(RPW // L):
        def row_body(j, carry, g=g):
            dotv, sqv = carry
            m = g * L + j
            dotc = jnp.zeros((L,), jnp.float32)
            sqc = jnp.zeros((L,), jnp.float32)
            for c in range(NCHUNK):
                acc = slab_v[m, pl.ds(c * L, L)]
                for a in range(1, A):
                    acc = acc + slab_v[m, pl.ds(a * D + c * L, L)]
                dotc = dotc + acc * q_v[pl.ds(c * L, L)]
                sqc = sqc + acc * acc
            lane = lanes == j
            return (jnp.where(lane, _splat_sum(dotc), dotv),
                    jnp.where(lane, _splat_sum(sqc), sqv))

        zero = jnp.zeros((L,), jnp.float32)
        dotv, sqv = lax.fori_loop(0, L, row_body, (zero, zero))
        dot_v[pl.ds(g * L, L)] = dotv
        sq_v[pl.ds(g * L, L)] = sqv

    pltpu.sync_copy(dot_v, dot_hbm.at[pl.ds(r0, RPW)])
    pltpu.sync_copy(sq_v, sq_hbm.at[pl.ds(r0, RPW)])


def _epi_body(dot_ref, sq_ref, q_ref, bank_ref,
              sims_ref, tv_ref, ti_ref, best_ref, sem):
    q = q_ref[...]
    qn = jnp.maximum(jnp.sqrt(jnp.sum(q * q)), 1e-8)
    dot = dot_ref[...]
    sq = sq_ref[...]
    norm = jnp.maximum(jnp.sqrt(sq) * (1.0 / A), 1e-8)
    sims = (dot * (1.0 / A)) / (norm * qn)
    sims_ref[...] = sims

    flat = (lax.broadcasted_iota(jnp.int32, (8, 128), 0) * 128
            + lax.broadcasted_iota(jnp.int32, (8, 128), 1))
    neg = jnp.float32(-jnp.inf)
    s = jnp.where(flat < M, sims, neg)
    picks = []
    for _ in range(K):
        gmax = jnp.max(s)
        gidx = jnp.min(jnp.where(s == gmax, flat, jnp.int32(2**30)))
        picks.append((gmax, gidx))
        s = jnp.where(flat == gidx, neg, s)

    tv = jnp.full((8, 128), 0.0, jnp.float32)
    ti = jnp.full((8, 128), 0, jnp.int32)
    for p, (gv, gi) in enumerate(picks):
        tv = jnp.where(flat == p, gv, tv)
        ti = jnp.where(flat == p, gi, ti)
    tv_ref[...] = tv
    ti_ref[...] = ti

    cp = pltpu.make_async_copy(bank_ref.at[picks[0][1]], best_ref, sem)
    cp.start()
    cp.wait()


_epi_call = pl.pallas_call(
    _epi_body,
    out_shape=(
        jax.ShapeDtypeStruct((8, 128), jnp.float32),
        jax.ShapeDtypeStruct((8, 128), jnp.float32),
        jax.ShapeDtypeStruct((8, 128), jnp.int32),
        jax.ShapeDtypeStruct((ROW // 128, 128), jnp.float32),
    ),
    in_specs=[
        pl.BlockSpec(memory_space=pltpu.MemorySpace.VMEM),
        pl.BlockSpec(memory_space=pltpu.MemorySpace.VMEM),
        pl.BlockSpec(memory_space=pltpu.MemorySpace.VMEM),
        pl.BlockSpec(memory_space=pl.ANY),
    ],
    out_specs=(
        pl.BlockSpec(memory_space=pltpu.MemorySpace.VMEM),
        pl.BlockSpec(memory_space=pltpu.MemorySpace.VMEM),
        pl.BlockSpec(memory_space=pltpu.MemorySpace.VMEM),
        pl.BlockSpec(memory_space=pltpu.MemorySpace.VMEM),
    ),
    scratch_shapes=[pltpu.SemaphoreType.DMA],
)


def kernel(query_embedding, memory_bank, k):
    bank2 = memory_bank.reshape(M, ROW)
    dotp, sqp = _dot_kernel(query_embedding, bank2)
    sims8, tv, ti, best = _epi_call(
        dotp.reshape(8, 128), sqp.reshape(8, 128),
        query_embedding.reshape(2, 128),
        memory_bank.reshape(M, ROW // 128, 128))
    return (sims8.reshape(MP)[:M],
            tv.reshape(MP)[:K],
            ti.reshape(MP)[:K],
            best.reshape(A, D))


# speedup vs baseline: 1.0525x; 1.0525x over previous
import functools

import jax
import jax.numpy as jnp
from jax import lax
from jax.experimental import pallas as pl
from jax.experimental.pallas import tpu as pltpu
from jax.experimental.pallas import tpu_sc as plsc

M = 1000
A = 7
D = 256
K = 3
L = 16
ROW = A * D
MP = 1024
NW = 32
RPW = MP // NW
LAST_ROWS = M - (NW - 1) * RPW
NCHUNK = D // L

_GATHER_DN = lax.GatherDimensionNumbers(
    offset_dims=(), collapsed_slice_dims=(0,), start_index_map=(0,))


def _lane_gather(x, idx):
    return lax.gather(x, idx[:, None], _GATHER_DN, slice_sizes=(1,),
                      mode=lax.GatherScatterMode.PROMISE_IN_BOUNDS)


def _splat_sum(x):
    lanes = lax.iota(jnp.int32, L)
    for step in (1, 2, 4, 8):
        x = x + _lane_gather(x, lanes ^ step)
    return x


_MESH = plsc.VectorSubcoreMesh(core_axis_name="c", subcore_axis_name="s")


@functools.partial(
    pl.kernel,
    out_type=(
        jax.ShapeDtypeStruct((MP,), jnp.float32),
        jax.ShapeDtypeStruct((MP,), jnp.float32),
    ),
    mesh=_MESH,
    scratch_types=[
        pltpu.VMEM((RPW, ROW), jnp.float32),
        pltpu.VMEM((D,), jnp.float32),
        pltpu.VMEM((RPW,), jnp.float32),
        pltpu.VMEM((RPW,), jnp.float32),
    ],
)
def _dot_kernel(q_hbm, bank_hbm, dot_hbm, sq_hbm, slab_v, q_v, dot_v, sq_v):
    wid = lax.axis_index("s") * 2 + lax.axis_index("c")
    r0 = wid * RPW
    pltpu.sync_copy(q_hbm, q_v)

    @pl.when(wid < NW - 1)
    def _():
        pltpu.sync_copy(bank_hbm.at[pl.ds(r0, RPW)], slab_v)

    @pl.when(wid == NW - 1)
    def _():
        pltpu.sync_copy(bank_hbm.at[pl.ds(r0, LAST_ROWS)],
                        slab_v.at[pl.ds(0, LAST_ROWS)])

    lanes = lax.iota(jnp.int32, L)
    for g in range(RPW // L):
        def row_body(j, carry, g=g):
            dotv, sqv = carry
            m = g * L + j
            dotc = jnp.zeros((L,), jnp.float32)
            sqc = jnp.zeros((L,), jnp.float32)
            for c in range(NCHUNK):
                acc = slab_v[m, pl.ds(c * L, L)]
                for a in range(1, A):
                    acc = acc + slab_v[m, pl.ds(a * D + c * L, L)]
                dotc = dotc + acc * q_v[pl.ds(c * L, L)]
                sqc = sqc + acc * acc
            lane = lanes == j
            return (jnp.where(lane, _splat_sum(dotc), dotv),
                    jnp.where(lane, _splat_sum(sqc), sqv))

        zero = jnp.zeros((L,), jnp.float32)
        dotv, sqv = lax.fori_loop(0, L, row_body, (zero, zero))
        dot_v[pl.ds(g * L, L)] = dotv
        sq_v[pl.ds(g * L, L)] = sqv

    pltpu.sync_copy(dot_v, dot_hbm.at[pl.ds(r0, RPW)])
    pltpu.sync_copy(sq_v, sq_hbm.at[pl.ds(r0, RPW)])


def _epi_body(dot_ref, sq_ref, q_ref, bank_ref,
              sims_ref, tv_ref, ti_ref, best_ref, sem):
    q = q_ref[...]
    qn = jnp.maximum(jnp.sqrt(jnp.sum(q * q)), 1e-8)
    dot = dot_ref[...]
    sq = sq_ref[...]
    norm = jnp.maximum(jnp.sqrt(sq) * (1.0 / A), 1e-8)
    sims = (dot * (1.0 / A)) / (norm * qn)
    sims_ref[...] = sims

    flat = (lax.broadcasted_iota(jnp.int32, (8, 128), 0) * 128
            + lax.broadcasted_iota(jnp.int32, (8, 128), 1))
    neg = jnp.float32(-jnp.inf)
    s = jnp.where(flat < M, sims, neg)
    picks = []
    for _ in range(K):
        gmax = jnp.max(s)
        gidx = jnp.min(jnp.where(s == gmax, flat, jnp.int32(2**30)))
        picks.append((gmax, gidx))
        s = jnp.where(flat == gidx, neg, s)

    tv = jnp.full((8, 128), 0.0, jnp.float32)
    ti = jnp.full((8, 128), 0, jnp.int32)
    for p, (gv, gi) in enumerate(picks):
        tv = jnp.where(flat == p, gv, tv)
        ti = jnp.where(flat == p, gi, ti)
    tv_ref[...] = tv
    ti_ref[...] = ti

    cp = pltpu.make_async_copy(bank_ref.at[picks[0][1]], best_ref, sem)
    cp.start()
    cp.wait()


_EPI_VMEM = pl.BlockSpec(memory_space=pltpu.MemorySpace.VMEM)


_epi_call = pl.pallas_call(
    _epi_body,
    out_shape=(
        jax.ShapeDtypeStruct((8, 128), jnp.float32),
        jax.ShapeDtypeStruct((8, 128), jnp.float32),
        jax.ShapeDtypeStruct((8, 128), jnp.int32),
        jax.ShapeDtypeStruct((A, D), jnp.float32),
    ),
    in_specs=[_EPI_VMEM, _EPI_VMEM, _EPI_VMEM,
              pl.BlockSpec(memory_space=pl.ANY)],
    out_specs=(_EPI_VMEM, _EPI_VMEM, _EPI_VMEM, _EPI_VMEM),
    scratch_shapes=[pltpu.SemaphoreType.DMA],
)


def kernel(query_embedding, memory_bank, k):
    bank2 = memory_bank.reshape(M, ROW)
    dotp, sqp = _dot_kernel(query_embedding, bank2)
    sims8, tv, ti, best = _epi_call(
        dotp.reshape(8, 128), sqp.reshape(8, 128),
        query_embedding.reshape(2, 128),
        memory_bank)
    return (sims8.reshape(MP)[:M],
            tv.reshape(MP)[:K],
            ti.reshape(MP)[:K],
            best)


# speedup vs baseline: 1.1822x; 1.1232x over previous
import functools

import jax
import jax.numpy as jnp
from jax import lax
from jax.experimental import pallas as pl
from jax.experimental.pallas import tpu as pltpu
from jax.experimental.pallas import tpu_sc as plsc

M = 1000
A = 7
D = 256
K = 3
L = 16
MP = 1024
NW = 32
RPW = MP // NW
LAST_ROWS = M - (NW - 1) * RPW
NCHUNK = D // L

_GATHER_DN = lax.GatherDimensionNumbers(
    offset_dims=(), collapsed_slice_dims=(0,), start_index_map=(0,))


def _lane_gather(x, idx):
    return lax.gather(x, idx[:, None], _GATHER_DN, slice_sizes=(1,),
                      mode=lax.GatherScatterMode.PROMISE_IN_BOUNDS)


def _splat_sum(x):
    lanes = lax.iota(jnp.int32, L)
    for step in (1, 2, 4, 8):
        x = x + _lane_gather(x, lanes ^ step)
    return x


def _sqrtv(x):
    xs = x + 1e-30
    xi = plsc.bitcast(xs, jnp.int32)
    r = plsc.bitcast(jnp.int32(0x5F3759DF) - lax.shift_right_logical(xi, 1),
                     jnp.float32)
    for _ in range(3):
        r = r * (1.5 - 0.5 * xs * r * r)
    return xs * r


_MESH = plsc.VectorSubcoreMesh(core_axis_name="c", subcore_axis_name="s")


@functools.partial(
    pl.kernel,
    out_type=(
        jax.ShapeDtypeStruct((MP,), jnp.float32),
        jax.ShapeDtypeStruct((MP,), jnp.float32),
    ),
    mesh=_MESH,
    scratch_types=[
        pltpu.VMEM((RPW, A, D), jnp.float32),
        pltpu.VMEM((D,), jnp.float32),
        pltpu.VMEM((RPW,), jnp.float32),
        pltpu.VMEM((RPW,), jnp.float32),
    ],
    compiler_params=pltpu.CompilerParams(use_tc_tiling_on_sc=True),
)
def _dot_kernel(q_hbm, bank_hbm, dot_hbm, sq_hbm, slab_v, q_v, dot_v, sq_v):
    wid = lax.axis_index("s") * 2 + lax.axis_index("c")
    r0 = wid * RPW
    pltpu.sync_copy(q_hbm, q_v)

    @pl.when(wid < NW - 1)
    def _():
        pltpu.sync_copy(bank_hbm.at[pl.ds(r0, RPW)], slab_v)

    @pl.when(wid == NW - 1)
    def _():
        pltpu.sync_copy(bank_hbm.at[pl.ds(r0, LAST_ROWS)],
                        slab_v.at[pl.ds(0, LAST_ROWS)])

    lanes = lax.iota(jnp.int32, L)
    for g in range(RPW // L):
        def row_body(j, carry, g=g):
            dotv, sqv = carry
            m = g * L + j
            dotc = jnp.zeros((L,), jnp.float32)
            sqc = jnp.zeros((L,), jnp.float32)
            for c in range(NCHUNK):
                acc = slab_v[m, 0, pl.ds(c * L, L)]
                for a in range(1, A):
                    acc = acc + slab_v[m, a, pl.ds(c * L, L)]
                dotc = dotc + acc * q_v[pl.ds(c * L, L)]
                sqc = sqc + acc * acc
            lane = lanes == j
            return (jnp.where(lane, _splat_sum(dotc), dotv),
                    jnp.where(lane, _splat_sum(sqc), sqv))

        zero = jnp.zeros((L,), jnp.float32)
        dotv, sqv = lax.fori_loop(0, L, row_body, (zero, zero))
        dot_v[pl.ds(g * L, L)] = dotv
        sq_v[pl.ds(g * L, L)] = sqv

    pltpu.sync_copy(dot_v, dot_hbm.at[pl.ds(r0, RPW)])
    pltpu.sync_copy(sq_v, sq_hbm.at[pl.ds(r0, RPW)])


@functools.partial(
    pl.kernel,
    out_type=(
        jax.ShapeDtypeStruct((MP,), jnp.float32),
        jax.ShapeDtypeStruct((L,), jnp.float32),
        jax.ShapeDtypeStruct((L,), jnp.int32),
        jax.ShapeDtypeStruct((A, D), jnp.float32),
    ),
    mesh=_MESH,
    scratch_types=[
        pltpu.VMEM((MP,), jnp.float32),
        pltpu.VMEM((MP,), jnp.float32),
        pltpu.VMEM((MP,), jnp.float32),
        pltpu.VMEM((D,), jnp.float32),
        pltpu.VMEM((L,), jnp.float32),
        pltpu.VMEM((L,), jnp.int32),
        pltpu.VMEM((A, D), jnp.float32),
    ],
    compiler_params=pltpu.CompilerParams(
        use_tc_tiling_on_sc=True, needs_layout_passes=False),
)
def _top_kernel(dot_hbm, sq_hbm, q_hbm, bank_hbm,
                sims_hbm, tv_hbm, ti_hbm, best_hbm,
                dot_v, sq_v, sims_v, q_v, tv_v, ti_v, best_v):
    wid = lax.axis_index("s") * 2 + lax.axis_index("c")

    @pl.when(wid == 0)
    def _():
        pltpu.sync_copy(dot_hbm, dot_v)
        pltpu.sync_copy(sq_hbm, sq_v)
        pltpu.sync_copy(q_hbm, q_v)

        lanes = lax.iota(jnp.int32, L)
        qq = jnp.zeros((L,), jnp.float32)
        for c in range(NCHUNK):
            qc = q_v[pl.ds(c * L, L)]
            qq = qq + qc * qc
        qnv = jnp.maximum(_sqrtv(_splat_sum(qq)), 1e-8)

        inv_a = jnp.float32(1.0 / A)
        neg = jnp.float32(-jnp.inf)

        def chunk_body(i, carry):
            d = dot_v[pl.ds(i * L, L)]
            s = sq_v[pl.ds(i * L, L)]
            norm = jnp.maximum(_sqrtv(s) * inv_a, 1e-8)
            sims = (d * inv_a) / (norm * qnv)
            gidx = lanes + i * L
            sims = jnp.where(gidx < M, sims, neg)
            sims_v[pl.ds(i * L, L)] = sims
            return carry

        lax.fori_loop(0, MP // L, chunk_body, jnp.int32(0))

        def select_next(prev):
            def eff(i):
                v = sims_v[pl.ds(i * L, L)]
                if prev is not None:
                    gv, gi = prev
                    gidx = lanes + i * L
                    keep = (v < gv) | ((v == gv) & (gidx > gi))
                    v = jnp.where(keep, v, neg)
                return v

            m = lax.fori_loop(
                0, MP // L, lambda i, mm: jnp.maximum(mm, eff(i)),
                jnp.full((L,), neg))
            gmax = jnp.max(m)

            def arg_body(i, best):
                cand = jnp.where(eff(i) == gmax, lanes + i * L,
                                 jnp.int32(2**30))
                return jnp.minimum(best, cand)

            bi = lax.fori_loop(0, MP // L, arg_body,
                               jnp.full((L,), 2**30, jnp.int32))
            return gmax, jnp.min(bi)

        p0 = select_next(None)
        p1 = select_next(p0)
        p2 = select_next(p1)

        tv = jnp.where(lanes == 0, p0[0],
                       jnp.where(lanes == 1, p1[0],
                                 jnp.where(lanes == 2, p2[0],
                                           jnp.float32(0.0))))
        ti = jnp.where(lanes == 0, p0[1],
                       jnp.where(lanes == 1, p1[1],
                                 jnp.where(lanes == 2, p2[1],
                                           jnp.int32(0))))
        tv_v[...] = tv
        ti_v[...] = ti
        pltpu.sync_copy(sims_v, sims_hbm)
        pltpu.sync_copy(tv_v, tv_hbm)
        pltpu.sync_copy(ti_v, ti_hbm)
        pltpu.sync_copy(bank_hbm.at[p0[1]], best_v)
        pltpu.sync_copy(best_v, best_hbm)


def kernel(query_embedding, memory_bank, k):
    dotp, sqp = _dot_kernel(query_embedding, memory_bank)
    sims_p, tv, ti, best = _top_kernel(dotp, sqp, query_embedding,
                                       memory_bank)
    return (sims_p[:M], tv[:K], ti[:K], best)
